# Initial kernel scaffold; baseline (speedup 1.0000x reference)
import jax
import jax.numpy as jnp
from jax.experimental import pallas as pl


def kernel(pcs):
    raise NotImplementedError("write your pallas kernel here")


# speedup vs baseline: 5.0015x; 5.0015x over previous
import functools

import jax
import jax.numpy as jnp
from jax import lax
from jax.experimental import pallas as pl
from jax.experimental.pallas import tpu as pltpu
from jax.experimental.pallas import tpu_sc as plsc

KNN = 10
NSEEDS = 20
NPTS = 16384
NB = 16
LANES = 16
STEPS = NPTS // LANES
SEEDS_PER_ROLE = NSEEDS // 2
OUT_W = NSEEDS * LANES


def _sc_body(x_hbm, y_hbm, z_hbm, out_hbm, xs, ys, zs, dist, seedc, obuf):
    c = lax.axis_index("c")
    s = lax.axis_index("s")
    b = 8 * c + lax.rem(s, 8)
    role = s // 8

    pltpu.sync_copy(x_hbm.at[pl.ds(b * NPTS, NPTS)], xs)
    pltpu.sync_copy(y_hbm.at[pl.ds(b * NPTS, NPTS)], ys)
    pltpu.sync_copy(z_hbm.at[pl.ds(b * NPTS, NPTS)], zs)

    iota = lax.iota(jnp.int32, LANES)
    neginf = jnp.full((LANES,), -jnp.inf, jnp.float32)
    posinf = jnp.full((LANES,), jnp.inf, jnp.float32)
    zero_idx = jnp.zeros((LANES,), jnp.int32)

    cx0 = plsc.load_gather(xs, [zero_idx])
    cy0 = plsc.load_gather(ys, [zero_idx])
    cz0 = plsc.load_gather(zs, [zero_idx])

    cap = jnp.full((LANES,), 1e10, jnp.float32)

    def step0(i, c2):
        bestv, besti = c2
        base = i * LANES
        px = xs[pl.ds(base, LANES)]
        py = ys[pl.ds(base, LANES)]
        pz = zs[pl.ds(base, LANES)]
        dx = px - cx0
        dy = py - cy0
        dz = pz - cz0
        d2 = (dx * dx + dy * dy) + dz * dz
        nd = jnp.minimum(cap, d2)
        dist[pl.ds(base, LANES)] = nd
        upd = nd > bestv
        bestv = jnp.where(upd, nd, bestv)
        besti = jnp.where(upd, base + iota, besti)
        return bestv, besti

    bestv0, besti0 = lax.fori_loop(0, STEPS, step0, (neginf, zero_idx))
    m0 = jnp.max(bestv0)
    gidx0 = jnp.min(jnp.where(bestv0 == m0, besti0, jnp.int32(2**30)))
    gvec0 = jnp.full((LANES,), gidx0, jnp.int32)
    cx1 = plsc.load_gather(xs, [gvec0])
    cy1 = plsc.load_gather(ys, [gvec0])
    cz1 = plsc.load_gather(zs, [gvec0])

    def fps_round(r, carry):
        cx, cy, cz, sx0, sx1, sy0, sy1, sz0, sz1 = carry
        m0 = iota == r
        m1 = iota == (r - 16)
        sx0 = jnp.where(m0, cx, sx0)
        sx1 = jnp.where(m1, cx, sx1)
        sy0 = jnp.where(m0, cy, sy0)
        sy1 = jnp.where(m1, cy, sy1)
        sz0 = jnp.where(m0, cz, sz0)
        sz1 = jnp.where(m1, cz, sz1)

        def step(i, c2):
            bestv, besti = c2
            base = i * LANES
            px = xs[pl.ds(base, LANES)]
            py = ys[pl.ds(base, LANES)]
            pz = zs[pl.ds(base, LANES)]
            dx = px - cx
            dy = py - cy
            dz = pz - cz
            d2 = (dx * dx + dy * dy) + dz * dz
            old = dist[pl.ds(base, LANES)]
            nd = jnp.minimum(old, d2)
            dist[pl.ds(base, LANES)] = nd
            upd = nd > bestv
            bestv = jnp.where(upd, nd, bestv)
            besti = jnp.where(upd, base + iota, besti)
            return bestv, besti

        bestv, besti = lax.fori_loop(0, STEPS, step, (neginf, zero_idx))
        m = jnp.max(bestv)
        gidx = jnp.min(jnp.where(bestv == m, besti, jnp.int32(2**30)))
        gvec = jnp.full((LANES,), gidx, jnp.int32)
        cxn = plsc.load_gather(xs, [gvec])
        cyn = plsc.load_gather(ys, [gvec])
        czn = plsc.load_gather(zs, [gvec])
        return cxn, cyn, czn, sx0, sx1, sy0, sy1, sz0, sz1

    carry = (cx0, cy0, cz0, neginf, neginf, neginf, neginf, neginf, neginf)
    carry = lax.fori_loop(0, NSEEDS, fps_round, carry)
    _, _, _, sx0, sx1, sy0, sy1, sz0, sz1 = carry

    seedc[pl.ds(0, LANES)] = sx0
    seedc[pl.ds(16, LANES)] = sx1
    seedc[pl.ds(32, LANES)] = sy0
    seedc[pl.ds(48, LANES)] = sy1
    seedc[pl.ds(64, LANES)] = sz0
    seedc[pl.ds(80, LANES)] = sz1

    for jj in range(SEEDS_PER_ROLE):
        j = role * SEEDS_PER_ROLE + jj
        jvec = jnp.full((LANES,), 0, jnp.int32) + j
        sx = plsc.load_gather(seedc, [jvec])
        sy = plsc.load_gather(seedc, [jvec + 32])
        sz = plsc.load_gather(seedc, [jvec + 64])

        def merge(cu, d2):
            sk, _ = plsc.sort_key_val(d2, iota)
            rc = lax.rev(cu, (0,))
            mm = jnp.minimum(sk, rc)
            nc, _ = plsc.sort_key_val(mm, iota)
            return nc, jnp.max(nc)

        def s2step(i, c2):
            cur, thr = c2
            base = i * LANES
            px = xs[pl.ds(base, LANES)]
            py = ys[pl.ds(base, LANES)]
            pz = zs[pl.ds(base, LANES)]
            dx = px - sx
            dy = py - sy
            dz = pz - sz
            d2 = (dx * dx + dy * dy) + dz * dz
            hit = jnp.any(d2 < thr)
            cur, thr = lax.cond(
                hit,
                lambda cu, th, dd: merge(cu, dd),
                lambda cu, th, dd: (cu, th),
                cur, thr, d2,
            )
            return cur, thr

        cur, _ = lax.fori_loop(
            0, STEPS, s2step, (posinf, jnp.float32(jnp.inf))
        )
        obuf[pl.ds(jj * LANES, LANES)] = cur

    pltpu.sync_copy(
        obuf,
        out_hbm.at[pl.ds(b * OUT_W + role * SEEDS_PER_ROLE * LANES,
                         SEEDS_PER_ROLE * LANES)],
    )


_sc_topk = functools.partial(
    pl.kernel,
    mesh=plsc.VectorSubcoreMesh(core_axis_name="c", subcore_axis_name="s"),
    out_type=jax.ShapeDtypeStruct((NB * OUT_W,), jnp.float32),
    scratch_types=[
        pltpu.VMEM((NPTS,), jnp.float32),
        pltpu.VMEM((NPTS,), jnp.float32),
        pltpu.VMEM((NPTS,), jnp.float32),
        pltpu.VMEM((NPTS,), jnp.float32),
        pltpu.VMEM((96,), jnp.float32),
        pltpu.VMEM((SEEDS_PER_ROLE * LANES,), jnp.float32),
    ],
    compiler_params=pltpu.CompilerParams(needs_layout_passes=False),
)(_sc_body)


def _tc_body(t_ref, o_ref):
    t = t_ref[...]
    col = lax.broadcasted_iota(jnp.int32, (NB, OUT_W), 1)
    k = lax.rem(col, LANES)
    mask = (k >= 1) & (k <= KNN)
    sdist = jnp.sqrt(t + 1e-12)
    w = jnp.exp(-(sdist * sdist) * (1.0 / (0.01 ** 2)))
    val = jnp.where(mask, -sdist * w, 0.0)
    o_ref[...] = (jnp.sum(val) / NB).reshape(1, 1)


def kernel(pcs):
    x = pcs[:, :, 0].reshape(-1)
    y = pcs[:, :, 1].reshape(-1)
    z = pcs[:, :, 2].reshape(-1)
    t = _sc_topk(x, y, z).reshape(NB, OUT_W)
    loss = pl.pallas_call(
        _tc_body,
        out_shape=jax.ShapeDtypeStruct((1, 1), jnp.float32),
    )(t)
    return loss[0, 0]


# speedup vs baseline: 17.4016x; 3.4793x over previous
import functools

import jax
import jax.numpy as jnp
from jax import lax
from jax.experimental import pallas as pl
from jax.experimental.pallas import tpu as pltpu
from jax.experimental.pallas import tpu_sc as plsc

KNN = 10
NSEEDS = 20
NPTS = 16384
NB = 16
LANES = 16
HALF = NPTS // 2
STEPS_H = HALF // LANES
STEPS = NPTS // LANES
SEEDS_PER_ROLE = NSEEDS // 2
NTOP = KNN + 1
OUT_W = NSEEDS * LANES


def _sc_body(x_hbm, y_hbm, z_hbm, init_hbm, out_hbm, xs, ys, zs, dist, seedc,
             obuf, xbuf, ybuf, shared):
    c = lax.axis_index("c")
    s = lax.axis_index("s")
    b = 8 * c + lax.rem(s, 8)
    role = s // 8
    partner = lax.rem(s + 8, 16)

    pltpu.sync_copy(x_hbm.at[pl.ds(b * NPTS, NPTS)], xs)
    pltpu.sync_copy(y_hbm.at[pl.ds(b * NPTS, NPTS)], ys)
    pltpu.sync_copy(z_hbm.at[pl.ds(b * NPTS, NPTS)], zs)
    pltpu.sync_copy(init_hbm, dist)

    BIGI = jnp.int32(2**30)
    iota = lax.iota(jnp.int32, LANES)
    neginf = jnp.full((LANES,), -jnp.inf, jnp.float32)
    posinf = jnp.full((LANES,), jnp.inf, jnp.float32)
    zero_idx = jnp.zeros((LANES,), jnp.int32)
    base0 = role * HALF

    cx0 = plsc.load_gather(xs, [zero_idx])
    cy0 = plsc.load_gather(ys, [zero_idx])
    cz0 = plsc.load_gather(zs, [zero_idx])

    def halfpass(cx, cy, cz):
        def step(i, c2):
            bestv, besti = c2
            lbase = i * LANES
            gbase = base0 + lbase
            px = xs[pl.ds(gbase, LANES)]
            py = ys[pl.ds(gbase, LANES)]
            pz = zs[pl.ds(gbase, LANES)]
            dx = px - cx
            dy = py - cy
            dz = pz - cz
            d2 = (dx * dx + dy * dy) + dz * dz
            old = dist[pl.ds(lbase, LANES)]
            nd = jnp.minimum(old, d2)
            dist[pl.ds(lbase, LANES)] = nd
            upd = nd > bestv
            bestv = jnp.where(upd, nd, bestv)
            besti = jnp.where(upd, gbase + iota, besti)
            return bestv, besti

        return lax.fori_loop(0, STEPS_H, step, (neginf, zero_idx))

    def exchange(r, bestv, besti):
        xbuf[pl.ds(0, LANES)] = bestv
        xbuf[pl.ds(16, LANES)] = plsc.bitcast(besti, jnp.float32)
        slot = lax.rem(r, 2) * 512
        pltpu.sync_copy(xbuf, shared.at[pl.ds(slot + s * 32, 32)])
        plsc.subcore_barrier()
        pltpu.sync_copy(shared.at[pl.ds(slot + partner * 32, 32)], ybuf)
        pv = ybuf[pl.ds(0, LANES)]
        pi = plsc.bitcast(ybuf[pl.ds(16, LANES)], jnp.int32)
        m = jnp.maximum(jnp.max(bestv), jnp.max(pv))
        cand_s = jnp.min(jnp.where(bestv == m, besti, BIGI))
        cand_p = jnp.min(jnp.where(pv == m, pi, BIGI))
        gidx = jnp.minimum(cand_s, cand_p)
        gvec = jnp.full((LANES,), gidx, jnp.int32)
        cxn = plsc.load_gather(xs, [gvec])
        cyn = plsc.load_gather(ys, [gvec])
        czn = plsc.load_gather(zs, [gvec])
        return cxn, cyn, czn

    def fps_round(r, carry):
        cx, cy, cz, sx0, sx1, sy0, sy1, sz0, sz1 = carry
        m0 = iota == r
        m1 = iota == (r - 16)
        sx0 = jnp.where(m0, cx, sx0)
        sx1 = jnp.where(m1, cx, sx1)
        sy0 = jnp.where(m0, cy, sy0)
        sy1 = jnp.where(m1, cy, sy1)
        sz0 = jnp.where(m0, cz, sz0)
        sz1 = jnp.where(m1, cz, sz1)
        bestv, besti = halfpass(cx, cy, cz)
        cxn, cyn, czn = exchange(r, bestv, besti)
        return cxn, cyn, czn, sx0, sx1, sy0, sy1, sz0, sz1

    carry = (cx0, cy0, cz0,
             neginf, neginf, neginf, neginf, neginf, neginf)
    carry = lax.fori_loop(0, NSEEDS, fps_round, carry)
    _, _, _, sx0, sx1, sy0, sy1, sz0, sz1 = carry

    seedc[pl.ds(0, LANES)] = sx0
    seedc[pl.ds(16, LANES)] = sx1
    seedc[pl.ds(32, LANES)] = sy0
    seedc[pl.ds(48, LANES)] = sy1
    seedc[pl.ds(64, LANES)] = sz0
    seedc[pl.ds(80, LANES)] = sz1

    for jj in range(SEEDS_PER_ROLE):
        j = role * SEEDS_PER_ROLE + jj
        jvec = zero_idx + j
        sx = plsc.load_gather(seedc, [jvec])
        sy = plsc.load_gather(seedc, [jvec + 32])
        sz = plsc.load_gather(seedc, [jvec + 64])

        def s2step(i, ms):
            base = i * LANES
            px = xs[pl.ds(base, LANES)]
            py = ys[pl.ds(base, LANES)]
            pz = zs[pl.ds(base, LANES)]
            dx = px - sx
            dy = py - sy
            dz = pz - sz
            d2 = (dx * dx + dy * dy) + dz * dz
            out = [jnp.minimum(ms[0], d2)]
            for t in range(1, NTOP):
                out.append(jnp.minimum(ms[t], jnp.maximum(ms[t - 1], d2)))
            return tuple(out)

        ms = lax.fori_loop(0, STEPS, s2step, (posinf,) * NTOP)

        cur = posinf
        mlist = list(ms)
        for t in range(NTOP):
            g = jnp.min(mlist[0])
            cur = jnp.where(iota == t, g, cur)
            fidx = plsc.all_reduce_ffs(mlist[0] == g)
            rm = iota == fidx
            for u in range(NTOP - 1):
                mlist[u] = jnp.where(rm, mlist[u + 1], mlist[u])
            mlist[NTOP - 1] = jnp.where(rm, posinf, mlist[NTOP - 1])
        obuf[pl.ds(jj * LANES, LANES)] = cur

    pltpu.sync_copy(
        obuf,
        out_hbm.at[pl.ds(b * OUT_W + role * SEEDS_PER_ROLE * LANES,
                         SEEDS_PER_ROLE * LANES)],
    )


_sc_topk = functools.partial(
    pl.kernel,
    mesh=plsc.VectorSubcoreMesh(core_axis_name="c", subcore_axis_name="s"),
    out_type=jax.ShapeDtypeStruct((NB * OUT_W,), jnp.float32),
    scratch_types=[
        pltpu.VMEM((NPTS,), jnp.float32),
        pltpu.VMEM((NPTS,), jnp.float32),
        pltpu.VMEM((NPTS,), jnp.float32),
        pltpu.VMEM((HALF,), jnp.float32),
        pltpu.VMEM((96,), jnp.float32),
        pltpu.VMEM((SEEDS_PER_ROLE * LANES,), jnp.float32),
        pltpu.VMEM((32,), jnp.float32),
        pltpu.VMEM((32,), jnp.float32),
        pltpu.VMEM_SHARED((1024,), jnp.float32),
    ],
    compiler_params=pltpu.CompilerParams(needs_layout_passes=False),
)(_sc_body)


def _tc_body(t_ref, o_ref):
    t = t_ref[...]
    col = lax.broadcasted_iota(jnp.int32, (NB, OUT_W), 1)
    k = lax.rem(col, LANES)
    mask = (k >= 1) & (k <= KNN)
    sdist = jnp.sqrt(t + 1e-12)
    w = jnp.exp(-(sdist * sdist) * (1.0 / (0.01 ** 2)))
    val = jnp.where(mask, -sdist * w, 0.0)
    o_ref[...] = (jnp.sum(val) / NB).reshape(1, 1)


def kernel(pcs):
    x = pcs[:, :, 0].reshape(-1)
    y = pcs[:, :, 1].reshape(-1)
    z = pcs[:, :, 2].reshape(-1)
    init = jnp.full((HALF,), 1e10, jnp.float32)
    t = _sc_topk(x, y, z, init).reshape(NB, OUT_W)
    loss = pl.pallas_call(
        _tc_body,
        out_shape=jax.ShapeDtypeStruct((1, 1), jnp.float32),
    )(t)
    return loss[0, 0]
